# baseline (speedup 1.0000x reference)
import jax
import jax.numpy as jnp
from jax import lax
from jax.experimental import pallas as pl
from jax.experimental.pallas import tpu as pltpu

_DROP = 0.2
_SCALE = 1.0 / (1.0 - _DROP)
_S2 = _SCALE * _SCALE

_CR = 8
_DEPTH = 8


def _body(mu4_hbm, keep4_hbm, muc_hbm, keepc_hbm, sig_hbm,
          mu_out_hbm, sig_out_hbm,
          mu4_v, keep4_v, muc_v, keepc_v, colf_v, rowf_v,
          in_bufs, out_bufs, small_sems, in_sems, out_sems):
    n_chunks = sig_hbm.shape[0]
    chunks_per_b = mu4_hbm.shape[1] // _CR

    pltpu.make_async_copy(mu4_hbm, mu4_v, small_sems.at[0]).start()
    pltpu.make_async_copy(keep4_hbm, keep4_v, small_sems.at[1]).start()
    pltpu.make_async_copy(muc_hbm, muc_v, small_sems.at[2]).start()
    pltpu.make_async_copy(keepc_hbm, keepc_v, small_sems.at[3]).start()

    for d in range(_DEPTH):
        pltpu.make_async_copy(sig_hbm.at[d], in_bufs.at[d],
                              in_sems.at[d]).start()

    pltpu.make_async_copy(mu4_hbm, mu4_v, small_sems.at[0]).wait()
    pltpu.make_async_copy(keep4_hbm, keep4_v, small_sems.at[1]).wait()
    pltpu.make_async_copy(muc_hbm, muc_v, small_sems.at[2]).wait()
    pltpu.make_async_copy(keepc_hbm, keepc_v, small_sems.at[3]).wait()

    mu4 = mu4_v[...]
    keep4 = keep4_v[...]
    mu_scaled = mu4 * (_SCALE * keep4)
    mu4_v[...] = mu_scaled
    colf_v[...] = jnp.where(mu_scaled != 0.0, 1.0, 0.0)
    muc = muc_v[...]
    keepc = keepc_v[...]
    rowf_v[...] = jnp.where(muc * keepc != 0.0, _S2, 0.0)
    pltpu.make_async_copy(mu4_v, mu_out_hbm, small_sems.at[0]).start()

    def loop(i, carry):
        slot = lax.rem(i, _DEPTH)
        b = i // chunks_per_b
        pltpu.make_async_copy(sig_hbm.at[i], in_bufs.at[slot],
                              in_sems.at[slot]).wait()

        @pl.when(i >= _DEPTH)
        def _():
            pltpu.make_async_copy(out_bufs.at[slot], sig_out_hbm.at[i],
                                  out_sems.at[slot]).wait()

        sig = in_bufs[slot]
        rowf = rowf_v[i]
        colf = colf_v[b]
        out_bufs[slot] = sig * rowf[:, None, :] * colf[None, :, :]
        pltpu.make_async_copy(out_bufs.at[slot], sig_out_hbm.at[i],
                              out_sems.at[slot]).start()

        @pl.when(i + _DEPTH < n_chunks)
        def _():
            pltpu.make_async_copy(sig_hbm.at[i + _DEPTH], in_bufs.at[slot],
                                  in_sems.at[slot]).start()
        return carry

    lax.fori_loop(0, n_chunks, loop, 0)

    for d in range(_DEPTH):
        i = n_chunks - _DEPTH + d
        slot = i % _DEPTH
        pltpu.make_async_copy(out_bufs.at[slot], sig_out_hbm.at[i],
                              out_sems.at[slot]).wait()
    pltpu.make_async_copy(mu4_v, mu_out_hbm, small_sems.at[0]).wait()


def kernel(mu_in, Sigma_in):
    B, H, W, C = mu_in.shape
    HW = H * W
    n_chunks = B * HW // _CR
    keep = jax.random.bernoulli(jax.random.key(42), 1.0 - _DROP, mu_in.shape)
    keepf = keep.astype(jnp.float32).reshape(B, HW, C)
    mu3 = mu_in.reshape(B, HW, C)
    mu_chunked = mu3.reshape(n_chunks, _CR, C)
    keep_chunked = keepf.reshape(n_chunks, _CR, C)
    sig_chunked = Sigma_in.reshape(n_chunks, _CR, HW, C)

    hbm = pl.BlockSpec(memory_space=pltpu.MemorySpace.HBM)
    mu_out3, sig_out = pl.pallas_call(
        _body,
        in_specs=[hbm] * 5,
        out_specs=[hbm, hbm],
        out_shape=[
            jax.ShapeDtypeStruct((B, HW, C), jnp.float32),
            jax.ShapeDtypeStruct((n_chunks, _CR, HW, C), jnp.float32),
        ],
        scratch_shapes=[
            pltpu.VMEM((B, HW, C), jnp.float32),
            pltpu.VMEM((B, HW, C), jnp.float32),
            pltpu.VMEM((n_chunks, _CR, C), jnp.float32),
            pltpu.VMEM((n_chunks, _CR, C), jnp.float32),
            pltpu.VMEM((B, HW, C), jnp.float32),
            pltpu.VMEM((n_chunks, _CR, C), jnp.float32),
            pltpu.VMEM((_DEPTH, _CR, HW, C), jnp.float32),
            pltpu.VMEM((_DEPTH, _CR, HW, C), jnp.float32),
            pltpu.SemaphoreType.DMA((4,)),
            pltpu.SemaphoreType.DMA((_DEPTH,)),
            pltpu.SemaphoreType.DMA((_DEPTH,)),
        ],
    )(mu3, keepf, mu_chunked, keep_chunked, sig_chunked)

    return mu_out3.reshape(B, H, W, C), sig_out.reshape(B, HW, HW, C)


# speedup vs baseline: 1.0037x; 1.0037x over previous
import jax
import jax.numpy as jnp
from jax import lax
from jax.experimental import pallas as pl
from jax.experimental.pallas import tpu as pltpu

_DROP = 0.2
_SCALE = 1.0 / (1.0 - _DROP)
_S2 = _SCALE * _SCALE

_CR = 16
_DEPTH = 8


def _body(mu4_hbm, keep4_hbm, muc_hbm, keepc_hbm, sig_hbm,
          mu_out_hbm, sig_out_hbm,
          mu4_v, keep4_v, muc_v, keepc_v, colf_v, rowf_v,
          in_bufs, out_bufs, small_sems, in_sems, out_sems):
    n_chunks = sig_hbm.shape[0]
    chunks_per_b = mu4_hbm.shape[1] // _CR

    pltpu.make_async_copy(mu4_hbm, mu4_v, small_sems.at[0]).start()
    pltpu.make_async_copy(keep4_hbm, keep4_v, small_sems.at[1]).start()
    pltpu.make_async_copy(muc_hbm, muc_v, small_sems.at[2]).start()
    pltpu.make_async_copy(keepc_hbm, keepc_v, small_sems.at[3]).start()

    for d in range(_DEPTH):
        pltpu.make_async_copy(sig_hbm.at[d], in_bufs.at[d],
                              in_sems.at[d]).start()

    pltpu.make_async_copy(mu4_hbm, mu4_v, small_sems.at[0]).wait()
    pltpu.make_async_copy(keep4_hbm, keep4_v, small_sems.at[1]).wait()
    pltpu.make_async_copy(muc_hbm, muc_v, small_sems.at[2]).wait()
    pltpu.make_async_copy(keepc_hbm, keepc_v, small_sems.at[3]).wait()

    mu4 = mu4_v[...]
    keep4 = keep4_v[...]
    mu_scaled = mu4 * (_SCALE * keep4)
    mu4_v[...] = mu_scaled
    colf_v[...] = jnp.where(mu_scaled != 0.0, 1.0, 0.0)
    muc = muc_v[...]
    keepc = keepc_v[...]
    rowf_v[...] = jnp.where(muc * keepc != 0.0, _S2, 0.0)
    pltpu.make_async_copy(mu4_v, mu_out_hbm, small_sems.at[0]).start()

    def loop(i, carry):
        slot = lax.rem(i, _DEPTH)
        b = i // chunks_per_b
        pltpu.make_async_copy(sig_hbm.at[i], in_bufs.at[slot],
                              in_sems.at[slot]).wait()

        @pl.when(i >= _DEPTH)
        def _():
            pltpu.make_async_copy(out_bufs.at[slot], sig_out_hbm.at[i],
                                  out_sems.at[slot]).wait()

        sig = in_bufs[slot]
        rowf = rowf_v[i]
        colf = colf_v[b]
        out_bufs[slot] = sig * rowf[:, None, :] * colf[None, :, :]
        pltpu.make_async_copy(out_bufs.at[slot], sig_out_hbm.at[i],
                              out_sems.at[slot]).start()

        @pl.when(i + _DEPTH < n_chunks)
        def _():
            pltpu.make_async_copy(sig_hbm.at[i + _DEPTH], in_bufs.at[slot],
                                  in_sems.at[slot]).start()
        return carry

    lax.fori_loop(0, n_chunks, loop, 0)

    for d in range(_DEPTH):
        i = n_chunks - _DEPTH + d
        slot = i % _DEPTH
        pltpu.make_async_copy(out_bufs.at[slot], sig_out_hbm.at[i],
                              out_sems.at[slot]).wait()
    pltpu.make_async_copy(mu4_v, mu_out_hbm, small_sems.at[0]).wait()


def kernel(mu_in, Sigma_in):
    B, H, W, C = mu_in.shape
    HW = H * W
    n_chunks = B * HW // _CR
    keep = jax.random.bernoulli(jax.random.key(42), 1.0 - _DROP, mu_in.shape)
    keepf = keep.astype(jnp.float32).reshape(B, HW, C)
    mu3 = mu_in.reshape(B, HW, C)
    mu_chunked = mu3.reshape(n_chunks, _CR, C)
    keep_chunked = keepf.reshape(n_chunks, _CR, C)
    sig_chunked = Sigma_in.reshape(n_chunks, _CR, HW, C)

    hbm = pl.BlockSpec(memory_space=pltpu.MemorySpace.HBM)
    mu_out3, sig_out = pl.pallas_call(
        _body,
        in_specs=[hbm] * 5,
        out_specs=[hbm, hbm],
        out_shape=[
            jax.ShapeDtypeStruct((B, HW, C), jnp.float32),
            jax.ShapeDtypeStruct((n_chunks, _CR, HW, C), jnp.float32),
        ],
        scratch_shapes=[
            pltpu.VMEM((B, HW, C), jnp.float32),
            pltpu.VMEM((B, HW, C), jnp.float32),
            pltpu.VMEM((n_chunks, _CR, C), jnp.float32),
            pltpu.VMEM((n_chunks, _CR, C), jnp.float32),
            pltpu.VMEM((B, HW, C), jnp.float32),
            pltpu.VMEM((n_chunks, _CR, C), jnp.float32),
            pltpu.VMEM((_DEPTH, _CR, HW, C), jnp.float32),
            pltpu.VMEM((_DEPTH, _CR, HW, C), jnp.float32),
            pltpu.SemaphoreType.DMA((4,)),
            pltpu.SemaphoreType.DMA((_DEPTH,)),
            pltpu.SemaphoreType.DMA((_DEPTH,)),
        ],
    )(mu3, keepf, mu_chunked, keep_chunked, sig_chunked)

    return mu_out3.reshape(B, H, W, C), sig_out.reshape(B, HW, HW, C)
